# Initial kernel scaffold; baseline (speedup 1.0000x reference)
import functools

import jax
import jax.numpy as jnp
from jax import lax
from jax.experimental import pallas as pl
from jax.experimental.pallas import tpu as pltpu
from jax.experimental.pallas import tpu_sc as plsc

N = 10000
D = 128
E = 320000
NS = 16
CHUNK = 128
EPT = E // NS
NFULL = EPT // CHUNK
TAIL = EPT - NFULL * CHUNK
RPT = N // NS
CW = 16
_ZCHUNKS = -(-RPT // CHUNK)


def _fill2d(ref, val):
    rows, cols = ref.shape

    def body(r, carry):
        for k in range(cols // 16):
            ref[r, pl.ds(k * 16, 16)] = jnp.full((16,), val, ref.dtype)
        return carry

    lax.fori_loop(0, rows, body, 0)


def _agg_body(x_hbm, src_c_hbm, dst_c_hbm, src_r_hbm, dst_r_hbm,
              sum_c_hbm, sum_r_hbm, cnt_c_hbm, cnt_r_hbm,
              acc, cacc, idx_v, didx_v, idx_t, didx_t, rows_v, zcnt_v,
              ones_v, sem):
    c = lax.axis_index("c")
    s = lax.axis_index("s")

    _fill2d(rows_v, 0.0)
    _fill2d(zcnt_v, 0.0)
    _fill2d(ones_v, 1.0)

    row0 = s * RPT
    for j in range(_ZCHUNKS):
        n = CHUNK if j < _ZCHUNKS - 1 else RPT - (_ZCHUNKS - 1) * CHUNK
        pltpu.sync_copy(rows_v.at[pl.ds(0, n)], acc.at[pl.ds(row0 + j * CHUNK, n)])
        pltpu.sync_copy(zcnt_v.at[pl.ds(0, n)], cacc.at[pl.ds(row0 + j * CHUNK, n)])
    plsc.subcore_barrier()

    def run(src_hbm, dst_hbm):
        base0 = s * EPT

        def chunk(g, carry):
            base = base0 + g * CHUNK
            pltpu.sync_copy(src_hbm.at[pl.ds(base, CHUNK)], idx_v)
            pltpu.sync_copy(dst_hbm.at[pl.ds(base, CHUNK)], didx_v)
            pltpu.async_copy(x_hbm.at[idx_v], rows_v, sem).wait()
            pltpu.sync_copy(rows_v, acc.at[didx_v], add=True)
            pltpu.sync_copy(ones_v, cacc.at[didx_v], add=True)
            return carry

        lax.fori_loop(0, NFULL, chunk, 0)

        base = base0 + NFULL * CHUNK
        pltpu.sync_copy(src_hbm.at[pl.ds(base, TAIL)], idx_t)
        pltpu.sync_copy(dst_hbm.at[pl.ds(base, TAIL)], didx_t)
        pltpu.async_copy(x_hbm.at[idx_t], rows_v.at[pl.ds(0, TAIL)], sem).wait()
        pltpu.sync_copy(rows_v.at[pl.ds(0, TAIL)], acc.at[didx_t], add=True)
        pltpu.sync_copy(ones_v.at[pl.ds(0, TAIL)], cacc.at[didx_t], add=True)

    @pl.when(c == 0)
    def _():
        run(src_c_hbm, dst_c_hbm)

    @pl.when(c == 1)
    def _():
        run(src_r_hbm, dst_r_hbm)

    plsc.subcore_barrier()

    for j in range(_ZCHUNKS):
        n = CHUNK if j < _ZCHUNKS - 1 else RPT - (_ZCHUNKS - 1) * CHUNK
        r = row0 + j * CHUNK

        @pl.when(c == 0)
        def _():
            pltpu.sync_copy(acc.at[pl.ds(r, n)], sum_c_hbm.at[pl.ds(r, n)])
            pltpu.sync_copy(cacc.at[pl.ds(r, n)], cnt_c_hbm.at[pl.ds(r, n)])

        @pl.when(c == 1)
        def _():
            pltpu.sync_copy(acc.at[pl.ds(r, n)], sum_r_hbm.at[pl.ds(r, n)])
            pltpu.sync_copy(cacc.at[pl.ds(r, n)], cnt_r_hbm.at[pl.ds(r, n)])


_agg = pl.kernel(
    _agg_body,
    out_type=(
        jax.ShapeDtypeStruct((N, D), jnp.float32),
        jax.ShapeDtypeStruct((N, D), jnp.float32),
        jax.ShapeDtypeStruct((N, CW), jnp.float32),
        jax.ShapeDtypeStruct((N, CW), jnp.float32),
    ),
    mesh=plsc.VectorSubcoreMesh(core_axis_name="c", subcore_axis_name="s"),
    scratch_types=(
        pltpu.VMEM_SHARED((N, D), jnp.float32),
        pltpu.VMEM_SHARED((N, CW), jnp.float32),
        pltpu.VMEM((CHUNK,), jnp.int32),
        pltpu.VMEM((CHUNK,), jnp.int32),
        pltpu.VMEM((TAIL,), jnp.int32),
        pltpu.VMEM((TAIL,), jnp.int32),
        pltpu.VMEM((CHUNK, D), jnp.float32),
        pltpu.VMEM((CHUNK, CW), jnp.float32),
        pltpu.VMEM((CHUNK, CW), jnp.float32),
        pltpu.SemaphoreType.DMA,
    ),
)


BLK = 2000


def _combine_body(relu, sum_ref, cnt_ref, xd_ref, wl_ref, bl_ref, wr_ref,
                  o_ref):
    cnt = jnp.maximum(cnt_ref[:, 0:1], 1.0)
    mean = sum_ref[...] / cnt
    acc = lax.dot_general(mean, wl_ref[...], (((1,), (1,)), ((), ())),
                          preferred_element_type=jnp.float32)
    acc = acc + lax.dot_general(xd_ref[...], wr_ref[...],
                                (((1,), (1,)), ((), ())),
                                preferred_element_type=jnp.float32)
    acc = acc + bl_ref[...]
    if relu:
        acc = jnp.maximum(acc, 0.0)
    o_ref[...] = acc


def _make_combine(relu):
    return pl.pallas_call(
        functools.partial(_combine_body, relu),
        out_shape=jax.ShapeDtypeStruct((N, D), jnp.float32),
        grid=(N // BLK,),
        in_specs=[
            pl.BlockSpec((BLK, D), lambda i: (i, 0)),
            pl.BlockSpec((BLK, CW), lambda i: (i, 0)),
            pl.BlockSpec((BLK, D), lambda i: (i, 0)),
            pl.BlockSpec((D, D), lambda i: (0, 0)),
            pl.BlockSpec((1, D), lambda i: (0, 0)),
            pl.BlockSpec((D, D), lambda i: (0, 0)),
        ],
        out_specs=pl.BlockSpec((BLK, D), lambda i: (i, 0)),
    )


_combine_relu = _make_combine(True)
_combine_lin = _make_combine(False)


def kernel(x_paper, x_author, edge_index_cites, edge_index_rev_writes,
           Wl1c, bl1c, Wr1c, Wl1r, bl1r, Wr1r,
           Wl2c, bl2c, Wr2c, Wl2r, bl2r, Wr2r):
    src_c = edge_index_cites[0].astype(jnp.int32)
    dst_c = edge_index_cites[1].astype(jnp.int32)
    src_r = edge_index_rev_writes[0].astype(jnp.int32)
    dst_r = edge_index_rev_writes[1].astype(jnp.int32)

    sum_c1, sum_r1, cnt_c, cnt_r = _agg(x_paper, src_c, dst_c, src_r, dst_r)
    p1 = _combine_relu(sum_c1, cnt_c, x_paper,
                       Wl1c, bl1c.reshape(1, D), Wr1c)
    a1 = _combine_relu(sum_r1, cnt_r, x_author,
                       Wl1r, bl1r.reshape(1, D), Wr1r)

    sum_c2, sum_r2, _, _ = _agg(p1, src_c, dst_c, src_r, dst_r)
    p2 = _combine_lin(sum_c2, cnt_c, p1, Wl2c, bl2c.reshape(1, D), Wr2c)
    a2 = _combine_lin(sum_r2, cnt_r, a1, Wl2r, bl2r.reshape(1, D), Wr2r)
    return (p2, a2)


# speedup vs baseline: 4.1857x; 4.1857x over previous
import functools

import jax
import jax.numpy as jnp
from jax import lax
from jax.experimental import pallas as pl
from jax.experimental.pallas import tpu as pltpu
from jax.experimental.pallas import tpu_sc as plsc

N = 10000
D = 128
E = 320000
NS = 16
CHUNK = 64
EPT = E // NS
NFULL = EPT // CHUNK
TAIL = EPT - NFULL * CHUNK
RPT = 624
RTAIL = N - NS * RPT
_ZSIZES = (64, 64, 64, 64, 64, 64, 64, 64, 64, 48)


def _fill2d(ref, val):
    rows, cols = ref.shape

    def body(r, carry):
        for k in range(cols // 16):
            ref[r, pl.ds(k * 16, 16)] = jnp.full((16,), val, ref.dtype)
        return carry

    lax.fori_loop(0, rows, body, 0)


def _fill1d(ref, val):
    n, = ref.shape
    for k in range(n // 16):
        ref[pl.ds(k * 16, 16)] = jnp.full((16,), val, ref.dtype)


def _agg_body(x_hbm, src_c_hbm, dst_c_hbm, src_r_hbm, dst_r_hbm,
              sum_c_hbm, sum_r_hbm, cnt_c_hbm, cnt_r_hbm,
              acc, cacc, idx_v, didx_v, idx_t, didx_t, rows_v, z1_v,
              ones_v, cbuf_v, sem):
    c = lax.axis_index("c")
    s = lax.axis_index("s")

    _fill2d(rows_v, 0.0)
    _fill1d(z1_v, 0.0)
    _fill1d(ones_v, 1.0)

    row0 = s * RPT
    off = 0
    for n in _ZSIZES:
        pltpu.sync_copy(rows_v.at[pl.ds(0, n)], acc.at[pl.ds(row0 + off, n)])
        pltpu.sync_copy(z1_v.at[pl.ds(0, n)], cacc.at[pl.ds(row0 + off, n)])
        off += n

    @pl.when(s == 0)
    def _():
        pltpu.sync_copy(rows_v.at[pl.ds(0, RTAIL)],
                        acc.at[pl.ds(NS * RPT, RTAIL)])
        pltpu.sync_copy(z1_v.at[pl.ds(0, RTAIL)],
                        cacc.at[pl.ds(NS * RPT, RTAIL)])

    plsc.subcore_barrier()

    def run(src_hbm, dst_hbm):
        base0 = s * EPT

        def chunk(g, carry):
            base = base0 + g * CHUNK
            pltpu.sync_copy(src_hbm.at[pl.ds(base, CHUNK)], idx_v)
            pltpu.sync_copy(dst_hbm.at[pl.ds(base, CHUNK)], didx_v)
            pltpu.async_copy(x_hbm.at[idx_v], rows_v, sem).wait()
            pltpu.sync_copy(rows_v, acc.at[didx_v], add=True)
            pltpu.sync_copy(ones_v, cacc.at[didx_v], add=True)
            return carry

        lax.fori_loop(0, NFULL, chunk, 0)

        base = base0 + NFULL * CHUNK
        pltpu.sync_copy(src_hbm.at[pl.ds(base, TAIL)], idx_t)
        pltpu.sync_copy(dst_hbm.at[pl.ds(base, TAIL)], didx_t)
        pltpu.async_copy(x_hbm.at[idx_t], rows_v.at[pl.ds(0, TAIL)], sem).wait()
        pltpu.sync_copy(rows_v.at[pl.ds(0, TAIL)], acc.at[didx_t], add=True)
        pltpu.sync_copy(ones_v.at[pl.ds(0, TAIL)], cacc.at[didx_t], add=True)

    @pl.when(c == 0)
    def _():
        run(src_c_hbm, dst_c_hbm)

    @pl.when(c == 1)
    def _():
        run(src_r_hbm, dst_r_hbm)

    plsc.subcore_barrier()

    def writeback(sum_hbm, cnt_hbm):
        o = 0
        for n in _ZSIZES:
            r = row0 + o
            pltpu.sync_copy(acc.at[pl.ds(r, n)], rows_v.at[pl.ds(0, n)])
            pltpu.sync_copy(rows_v.at[pl.ds(0, n)], sum_hbm.at[pl.ds(r, n)])
            pltpu.sync_copy(cacc.at[pl.ds(r, n)], cbuf_v.at[pl.ds(0, n)])
            pltpu.sync_copy(cbuf_v.at[pl.ds(0, n)], cnt_hbm.at[pl.ds(r, n)])
            o += n

        @pl.when(s == 0)
        def _():
            r = NS * RPT
            pltpu.sync_copy(acc.at[pl.ds(r, RTAIL)], rows_v.at[pl.ds(0, RTAIL)])
            pltpu.sync_copy(rows_v.at[pl.ds(0, RTAIL)], sum_hbm.at[pl.ds(r, RTAIL)])
            pltpu.sync_copy(cacc.at[pl.ds(r, RTAIL)], cbuf_v.at[pl.ds(0, RTAIL)])
            pltpu.sync_copy(cbuf_v.at[pl.ds(0, RTAIL)], cnt_hbm.at[pl.ds(r, RTAIL)])

    @pl.when(c == 0)
    def _():
        writeback(sum_c_hbm, cnt_c_hbm)

    @pl.when(c == 1)
    def _():
        writeback(sum_r_hbm, cnt_r_hbm)


_agg = pl.kernel(
    _agg_body,
    out_type=(
        jax.ShapeDtypeStruct((N, D), jnp.float32),
        jax.ShapeDtypeStruct((N, D), jnp.float32),
        jax.ShapeDtypeStruct((N,), jnp.float32),
        jax.ShapeDtypeStruct((N,), jnp.float32),
    ),
    mesh=plsc.VectorSubcoreMesh(core_axis_name="c", subcore_axis_name="s"),
    scratch_types=(
        pltpu.VMEM_SHARED((N, D), jnp.float32),
        pltpu.VMEM_SHARED((N,), jnp.float32),
        pltpu.VMEM((CHUNK,), jnp.int32),
        pltpu.VMEM((CHUNK,), jnp.int32),
        pltpu.VMEM((TAIL,), jnp.int32),
        pltpu.VMEM((TAIL,), jnp.int32),
        pltpu.VMEM((CHUNK, D), jnp.float32),
        pltpu.VMEM((CHUNK,), jnp.float32),
        pltpu.VMEM((CHUNK,), jnp.float32),
        pltpu.VMEM((CHUNK,), jnp.float32),
        pltpu.SemaphoreType.DMA,
    ),
)


BLK = 2000


def _combine_body(relu, sum_ref, cnt_ref, xd_ref, wl_ref, bl_ref, wr_ref,
                  o_ref):
    cnt = jnp.maximum(cnt_ref[...], 1.0)
    mean = sum_ref[...] / cnt
    acc = lax.dot_general(mean, wl_ref[...], (((1,), (1,)), ((), ())),
                          preferred_element_type=jnp.float32)
    acc = acc + lax.dot_general(xd_ref[...], wr_ref[...],
                                (((1,), (1,)), ((), ())),
                                preferred_element_type=jnp.float32)
    acc = acc + bl_ref[...]
    if relu:
        acc = jnp.maximum(acc, 0.0)
    o_ref[...] = acc


def _make_combine(relu):
    return pl.pallas_call(
        functools.partial(_combine_body, relu),
        out_shape=jax.ShapeDtypeStruct((N, D), jnp.float32),
        grid=(N // BLK,),
        in_specs=[
            pl.BlockSpec((BLK, D), lambda i: (i, 0)),
            pl.BlockSpec((BLK, 1), lambda i: (i, 0)),
            pl.BlockSpec((BLK, D), lambda i: (i, 0)),
            pl.BlockSpec((D, D), lambda i: (0, 0)),
            pl.BlockSpec((1, D), lambda i: (0, 0)),
            pl.BlockSpec((D, D), lambda i: (0, 0)),
        ],
        out_specs=pl.BlockSpec((BLK, D), lambda i: (i, 0)),
    )


_combine_relu = _make_combine(True)
_combine_lin = _make_combine(False)


def kernel(x_paper, x_author, edge_index_cites, edge_index_rev_writes,
           Wl1c, bl1c, Wr1c, Wl1r, bl1r, Wr1r,
           Wl2c, bl2c, Wr2c, Wl2r, bl2r, Wr2r):
    src_c = edge_index_cites[0].astype(jnp.int32)
    dst_c = edge_index_cites[1].astype(jnp.int32)
    src_r = edge_index_rev_writes[0].astype(jnp.int32)
    dst_r = edge_index_rev_writes[1].astype(jnp.int32)

    sum_c1, sum_r1, cnt_c, cnt_r = _agg(x_paper, src_c, dst_c, src_r, dst_r)
    cnt_c = cnt_c[:, None]
    cnt_r = cnt_r[:, None]
    p1 = _combine_relu(sum_c1, cnt_c, x_paper,
                       Wl1c, bl1c.reshape(1, D), Wr1c)
    a1 = _combine_relu(sum_r1, cnt_r, x_author,
                       Wl1r, bl1r.reshape(1, D), Wr1r)

    sum_c2, sum_r2, _, _ = _agg(p1, src_c, dst_c, src_r, dst_r)
    p2 = _combine_lin(sum_c2, cnt_c, p1, Wl2c, bl2c.reshape(1, D), Wr2c)
    a2 = _combine_lin(sum_r2, cnt_r, a1, Wl2r, bl2r.reshape(1, D), Wr2r)
    return (p2, a2)


# speedup vs baseline: 9.3977x; 2.2452x over previous
import functools

import jax
import jax.numpy as jnp
from jax import lax
from jax.experimental import pallas as pl
from jax.experimental.pallas import tpu as pltpu
from jax.experimental.pallas import tpu_sc as plsc

N = 10000
D = 128
E = 320000
NS = 16
CHUNK = 64
KPB = 8
EPT = E // NS
NFULL = EPT // CHUNK
NBLK = NFULL // KPB
TAIL = EPT - NFULL * CHUNK
RPT = 624
RTAIL = N - NS * RPT
_ZSIZES = (64, 64, 64, 64, 64, 64, 64, 64, 64, 48)


def _fill2d(ref, val):
    rows, cols = ref.shape

    def body(r, carry):
        for k in range(cols // 16):
            ref[r, pl.ds(k * 16, 16)] = jnp.full((16,), val, ref.dtype)
        return carry

    lax.fori_loop(0, rows, body, 0)


def _fill1d(ref, val):
    n, = ref.shape
    for k in range(n // 16):
        ref[pl.ds(k * 16, 16)] = jnp.full((16,), val, ref.dtype)


def _agg_body(x_hbm, src_c_hbm, dst_c_hbm, src_r_hbm, dst_r_hbm,
              sum_c_hbm, sum_r_hbm, cnt_c_hbm, cnt_r_hbm, *sc):
    idxs_v = sc[0:KPB]
    didxs_v = sc[KPB:2 * KPB]
    (acc, cacc, idx_t, didx_t, rows0_v, rows1_v, z1_v, ones_v, cbuf_v,
     isem, gsem0, gsem1) = sc[2 * KPB:]
    rows = (rows0_v, rows1_v)
    gsem = (gsem0, gsem1)
    rows_v = rows0_v
    c = lax.axis_index("c")
    s = lax.axis_index("s")

    _fill2d(rows_v, 0.0)
    _fill1d(z1_v, 0.0)
    _fill1d(ones_v, 1.0)

    row0 = s * RPT
    off = 0
    for n in _ZSIZES:
        pltpu.sync_copy(rows_v.at[pl.ds(0, n)], acc.at[pl.ds(row0 + off, n)])
        pltpu.sync_copy(z1_v.at[pl.ds(0, n)], cacc.at[pl.ds(row0 + off, n)])
        off += n

    @pl.when(s == 0)
    def _():
        pltpu.sync_copy(rows_v.at[pl.ds(0, RTAIL)],
                        acc.at[pl.ds(NS * RPT, RTAIL)])
        pltpu.sync_copy(z1_v.at[pl.ds(0, RTAIL)],
                        cacc.at[pl.ds(NS * RPT, RTAIL)])

    plsc.subcore_barrier()

    def run(src_hbm, dst_hbm):
        base0 = s * EPT

        def block(i, carry):
            base = base0 + i * (KPB * CHUNK)
            cps = []
            for k in range(KPB):
                cps.append(pltpu.async_copy(
                    src_hbm.at[pl.ds(base + k * CHUNK, CHUNK)], idxs_v[k], isem))
                cps.append(pltpu.async_copy(
                    dst_hbm.at[pl.ds(base + k * CHUNK, CHUNK)], didxs_v[k], isem))
            for cp in cps:
                cp.wait()
            g = [None] * KPB
            g[0] = pltpu.async_copy(x_hbm.at[idxs_v[0]], rows[0], gsem[0])
            for k in range(1, KPB):
                g[k] = pltpu.async_copy(x_hbm.at[idxs_v[k]], rows[k % 2],
                                        gsem[k % 2])
                g[k - 1].wait()
                pltpu.sync_copy(rows[(k - 1) % 2], acc.at[didxs_v[k - 1]],
                                add=True)
                pltpu.sync_copy(ones_v, cacc.at[didxs_v[k - 1]], add=True)
            g[KPB - 1].wait()
            pltpu.sync_copy(rows[(KPB - 1) % 2], acc.at[didxs_v[KPB - 1]],
                            add=True)
            pltpu.sync_copy(ones_v, cacc.at[didxs_v[KPB - 1]], add=True)
            return carry

        lax.fori_loop(0, NBLK, block, 0)

        base = base0 + NFULL * CHUNK
        pltpu.sync_copy(src_hbm.at[pl.ds(base, TAIL)], idx_t)
        pltpu.sync_copy(dst_hbm.at[pl.ds(base, TAIL)], didx_t)
        pltpu.async_copy(x_hbm.at[idx_t], rows[0].at[pl.ds(0, TAIL)],
                         gsem[0]).wait()
        pltpu.sync_copy(rows[0].at[pl.ds(0, TAIL)], acc.at[didx_t], add=True)
        pltpu.sync_copy(ones_v.at[pl.ds(0, TAIL)], cacc.at[didx_t], add=True)

    @pl.when(c == 0)
    def _():
        run(src_c_hbm, dst_c_hbm)

    @pl.when(c == 1)
    def _():
        run(src_r_hbm, dst_r_hbm)

    plsc.subcore_barrier()

    def writeback(sum_hbm, cnt_hbm):
        o = 0
        for n in _ZSIZES:
            r = row0 + o
            pltpu.sync_copy(acc.at[pl.ds(r, n)], rows_v.at[pl.ds(0, n)])
            pltpu.sync_copy(rows_v.at[pl.ds(0, n)], sum_hbm.at[pl.ds(r, n)])
            pltpu.sync_copy(cacc.at[pl.ds(r, n)], cbuf_v.at[pl.ds(0, n)])
            pltpu.sync_copy(cbuf_v.at[pl.ds(0, n)], cnt_hbm.at[pl.ds(r, n)])
            o += n

        @pl.when(s == 0)
        def _():
            r = NS * RPT
            pltpu.sync_copy(acc.at[pl.ds(r, RTAIL)], rows_v.at[pl.ds(0, RTAIL)])
            pltpu.sync_copy(rows_v.at[pl.ds(0, RTAIL)], sum_hbm.at[pl.ds(r, RTAIL)])
            pltpu.sync_copy(cacc.at[pl.ds(r, RTAIL)], cbuf_v.at[pl.ds(0, RTAIL)])
            pltpu.sync_copy(cbuf_v.at[pl.ds(0, RTAIL)], cnt_hbm.at[pl.ds(r, RTAIL)])

    @pl.when(c == 0)
    def _():
        writeback(sum_c_hbm, cnt_c_hbm)

    @pl.when(c == 1)
    def _():
        writeback(sum_r_hbm, cnt_r_hbm)


_agg = pl.kernel(
    _agg_body,
    out_type=(
        jax.ShapeDtypeStruct((N, D), jnp.float32),
        jax.ShapeDtypeStruct((N, D), jnp.float32),
        jax.ShapeDtypeStruct((N,), jnp.float32),
        jax.ShapeDtypeStruct((N,), jnp.float32),
    ),
    mesh=plsc.VectorSubcoreMesh(core_axis_name="c", subcore_axis_name="s"),
    scratch_types=(
        tuple(pltpu.VMEM((CHUNK,), jnp.int32) for _ in range(KPB))
        + tuple(pltpu.VMEM((CHUNK,), jnp.int32) for _ in range(KPB))
        + (
            pltpu.VMEM_SHARED((N, D), jnp.float32),
            pltpu.VMEM_SHARED((N,), jnp.float32),
            pltpu.VMEM((TAIL,), jnp.int32),
            pltpu.VMEM((TAIL,), jnp.int32),
            pltpu.VMEM((CHUNK, D), jnp.float32),
            pltpu.VMEM((CHUNK, D), jnp.float32),
            pltpu.VMEM((CHUNK,), jnp.float32),
            pltpu.VMEM((CHUNK,), jnp.float32),
            pltpu.VMEM((CHUNK,), jnp.float32),
            pltpu.SemaphoreType.DMA,
            pltpu.SemaphoreType.DMA,
            pltpu.SemaphoreType.DMA,
        )
    ),
)


BLK = 2000


def _combine_body(relu, sum_ref, cnt_ref, xd_ref, wl_ref, bl_ref, wr_ref,
                  o_ref):
    cnt = jnp.maximum(cnt_ref[...], 1.0)
    mean = sum_ref[...] / cnt
    acc = lax.dot_general(mean, wl_ref[...], (((1,), (1,)), ((), ())),
                          preferred_element_type=jnp.float32)
    acc = acc + lax.dot_general(xd_ref[...], wr_ref[...],
                                (((1,), (1,)), ((), ())),
                                preferred_element_type=jnp.float32)
    acc = acc + bl_ref[...]
    if relu:
        acc = jnp.maximum(acc, 0.0)
    o_ref[...] = acc


def _make_combine(relu):
    return pl.pallas_call(
        functools.partial(_combine_body, relu),
        out_shape=jax.ShapeDtypeStruct((N, D), jnp.float32),
        grid=(N // BLK,),
        in_specs=[
            pl.BlockSpec((BLK, D), lambda i: (i, 0)),
            pl.BlockSpec((BLK, 1), lambda i: (i, 0)),
            pl.BlockSpec((BLK, D), lambda i: (i, 0)),
            pl.BlockSpec((D, D), lambda i: (0, 0)),
            pl.BlockSpec((1, D), lambda i: (0, 0)),
            pl.BlockSpec((D, D), lambda i: (0, 0)),
        ],
        out_specs=pl.BlockSpec((BLK, D), lambda i: (i, 0)),
    )


_combine_relu = _make_combine(True)
_combine_lin = _make_combine(False)


def kernel(x_paper, x_author, edge_index_cites, edge_index_rev_writes,
           Wl1c, bl1c, Wr1c, Wl1r, bl1r, Wr1r,
           Wl2c, bl2c, Wr2c, Wl2r, bl2r, Wr2r):
    src_c = edge_index_cites[0].astype(jnp.int32)
    dst_c = edge_index_cites[1].astype(jnp.int32)
    src_r = edge_index_rev_writes[0].astype(jnp.int32)
    dst_r = edge_index_rev_writes[1].astype(jnp.int32)

    sum_c1, sum_r1, cnt_c, cnt_r = _agg(x_paper, src_c, dst_c, src_r, dst_r)
    cnt_c = cnt_c[:, None]
    cnt_r = cnt_r[:, None]
    p1 = _combine_relu(sum_c1, cnt_c, x_paper,
                       Wl1c, bl1c.reshape(1, D), Wr1c)
    a1 = _combine_relu(sum_r1, cnt_r, x_author,
                       Wl1r, bl1r.reshape(1, D), Wr1r)

    sum_c2, sum_r2, _, _ = _agg(p1, src_c, dst_c, src_r, dst_r)
    p2 = _combine_lin(sum_c2, cnt_c, p1, Wl2c, bl2c.reshape(1, D), Wr2c)
    a2 = _combine_lin(sum_r2, cnt_r, a1, Wl2r, bl2r.reshape(1, D), Wr2r)
    return (p2, a2)


# speedup vs baseline: 10.6699x; 1.1354x over previous
import functools

import jax
import jax.numpy as jnp
from jax import lax
from jax.experimental import pallas as pl
from jax.experimental.pallas import tpu as pltpu
from jax.experimental.pallas import tpu_sc as plsc

N = 10000
D = 128
E = 320000
NS = 16
CHUNK = 64
KPB = 8
NB = 3
EPT = E // NS
NFULL = EPT // CHUNK
NBLK = NFULL // KPB
TAIL = EPT - NFULL * CHUNK
RPT = 624
RTAIL = N - NS * RPT
_ZSIZES = (64, 64, 64, 64, 64, 64, 64, 64, 64, 48)


def _fill2d(ref, val):
    rows, cols = ref.shape

    def body(r, carry):
        for k in range(cols // 16):
            ref[r, pl.ds(k * 16, 16)] = jnp.full((16,), val, ref.dtype)
        return carry

    lax.fori_loop(0, rows, body, 0)


def _fill1d(ref, val):
    n, = ref.shape
    for k in range(n // 16):
        ref[pl.ds(k * 16, 16)] = jnp.full((16,), val, ref.dtype)


def _agg_body(with_counts, x_hbm, src_c_hbm, dst_c_hbm, src_r_hbm, dst_r_hbm,
              *args):
    if with_counts:
        sum_c_hbm, sum_r_hbm, cnt_c_hbm, cnt_r_hbm = args[:4]
        sc = args[4:]
    else:
        sum_c_hbm, sum_r_hbm = args[:2]
        sc = args[2:]
    idxs_v = sc[0:KPB]
    didxs_v = sc[KPB:2 * KPB]
    rows = sc[2 * KPB:2 * KPB + NB]
    gsem = sc[2 * KPB + NB:2 * KPB + 2 * NB]
    (acc, cacc, idx_t, didx_t, z1_v, ones_v, cbuf_v, isem, csem) = \
        sc[2 * KPB + 2 * NB:]
    rows_v = rows[0]
    c = lax.axis_index("c")
    s = lax.axis_index("s")

    _fill2d(rows_v, 0.0)
    _fill1d(z1_v, 0.0)
    _fill1d(ones_v, 1.0)

    row0 = s * RPT
    off = 0
    for n in _ZSIZES:
        pltpu.sync_copy(rows_v.at[pl.ds(0, n)], acc.at[pl.ds(row0 + off, n)])
        if with_counts:
            pltpu.sync_copy(z1_v.at[pl.ds(0, n)], cacc.at[pl.ds(row0 + off, n)])
        off += n

    @pl.when(s == 0)
    def _():
        pltpu.sync_copy(rows_v.at[pl.ds(0, RTAIL)],
                        acc.at[pl.ds(NS * RPT, RTAIL)])
        if with_counts:
            pltpu.sync_copy(z1_v.at[pl.ds(0, RTAIL)],
                            cacc.at[pl.ds(NS * RPT, RTAIL)])

    plsc.subcore_barrier()

    def run(src_hbm, dst_hbm):
        base0 = s * EPT

        def block(i, carry):
            base = base0 + i * (KPB * CHUNK)
            cps = []
            for k in range(KPB):
                cps.append(pltpu.async_copy(
                    src_hbm.at[pl.ds(base + k * CHUNK, CHUNK)], idxs_v[k], isem))
                cps.append(pltpu.async_copy(
                    dst_hbm.at[pl.ds(base + k * CHUNK, CHUNK)], didxs_v[k], isem))
            for cp in cps:
                cp.wait()
            g = [None] * KPB
            for k in range(NB - 1):
                g[k] = pltpu.async_copy(x_hbm.at[idxs_v[k]], rows[k % NB],
                                        gsem[k % NB])
            ccp = []
            for k in range(KPB):
                if k + NB - 1 < KPB:
                    j = k + NB - 1
                    g[j] = pltpu.async_copy(x_hbm.at[idxs_v[j]], rows[j % NB],
                                            gsem[j % NB])
                g[k].wait()
                pltpu.sync_copy(rows[k % NB], acc.at[didxs_v[k]], add=True)
                if with_counts:
                    ccp.append(pltpu.async_copy(ones_v, cacc.at[didxs_v[k]],
                                                csem, add=True))
            for cp in ccp:
                cp.wait()
            return carry

        lax.fori_loop(0, NBLK, block, 0)

        base = base0 + NFULL * CHUNK
        pltpu.sync_copy(src_hbm.at[pl.ds(base, TAIL)], idx_t)
        pltpu.sync_copy(dst_hbm.at[pl.ds(base, TAIL)], didx_t)
        pltpu.async_copy(x_hbm.at[idx_t], rows[0].at[pl.ds(0, TAIL)],
                         gsem[0]).wait()
        pltpu.sync_copy(rows[0].at[pl.ds(0, TAIL)], acc.at[didx_t], add=True)
        if with_counts:
            pltpu.sync_copy(ones_v.at[pl.ds(0, TAIL)], cacc.at[didx_t],
                            add=True)

    @pl.when(c == 0)
    def _():
        run(src_c_hbm, dst_c_hbm)

    @pl.when(c == 1)
    def _():
        run(src_r_hbm, dst_r_hbm)

    plsc.subcore_barrier()

    def writeback(sum_hbm, cnt_hbm):
        o = 0
        for n in _ZSIZES:
            r = row0 + o
            pltpu.sync_copy(acc.at[pl.ds(r, n)], rows_v.at[pl.ds(0, n)])
            pltpu.sync_copy(rows_v.at[pl.ds(0, n)], sum_hbm.at[pl.ds(r, n)])
            if with_counts:
                pltpu.sync_copy(cacc.at[pl.ds(r, n)], cbuf_v.at[pl.ds(0, n)])
                pltpu.sync_copy(cbuf_v.at[pl.ds(0, n)], cnt_hbm.at[pl.ds(r, n)])
            o += n

        @pl.when(s == 0)
        def _():
            r = NS * RPT
            pltpu.sync_copy(acc.at[pl.ds(r, RTAIL)], rows_v.at[pl.ds(0, RTAIL)])
            pltpu.sync_copy(rows_v.at[pl.ds(0, RTAIL)], sum_hbm.at[pl.ds(r, RTAIL)])
            if with_counts:
                pltpu.sync_copy(cacc.at[pl.ds(r, RTAIL)], cbuf_v.at[pl.ds(0, RTAIL)])
                pltpu.sync_copy(cbuf_v.at[pl.ds(0, RTAIL)], cnt_hbm.at[pl.ds(r, RTAIL)])

    @pl.when(c == 0)
    def _():
        writeback(sum_c_hbm, cnt_c_hbm if with_counts else None)

    @pl.when(c == 1)
    def _():
        writeback(sum_r_hbm, cnt_r_hbm if with_counts else None)


def _make_agg(with_counts):
    outs = (
        jax.ShapeDtypeStruct((N, D), jnp.float32),
        jax.ShapeDtypeStruct((N, D), jnp.float32),
    )
    if with_counts:
        outs = outs + (
            jax.ShapeDtypeStruct((N,), jnp.float32),
            jax.ShapeDtypeStruct((N,), jnp.float32),
        )
    return pl.kernel(
        functools.partial(_agg_body, with_counts),
        out_type=outs,
        mesh=plsc.VectorSubcoreMesh(core_axis_name="c", subcore_axis_name="s"),
        scratch_types=(
            tuple(pltpu.VMEM((CHUNK,), jnp.int32) for _ in range(KPB))
            + tuple(pltpu.VMEM((CHUNK,), jnp.int32) for _ in range(KPB))
            + tuple(pltpu.VMEM((CHUNK, D), jnp.float32) for _ in range(NB))
            + tuple(pltpu.SemaphoreType.DMA for _ in range(NB))
            + (
                pltpu.VMEM_SHARED((N, D), jnp.float32),
                pltpu.VMEM_SHARED((N,), jnp.float32),
                pltpu.VMEM((TAIL,), jnp.int32),
                pltpu.VMEM((TAIL,), jnp.int32),
                pltpu.VMEM((CHUNK,), jnp.float32),
                pltpu.VMEM((CHUNK,), jnp.float32),
                pltpu.VMEM((CHUNK,), jnp.float32),
                pltpu.SemaphoreType.DMA,
                pltpu.SemaphoreType.DMA,
            )
        ),
    )


_agg = _make_agg(True)
_agg_nc = _make_agg(False)


BLK = 2000


def _combine_body(relu, sum_ref, cnt_ref, xd_ref, wl_ref, bl_ref, wr_ref,
                  o_ref):
    cnt = jnp.maximum(cnt_ref[...], 1.0)
    mean = sum_ref[...] / cnt
    acc = lax.dot_general(mean, wl_ref[...], (((1,), (1,)), ((), ())),
                          preferred_element_type=jnp.float32)
    acc = acc + lax.dot_general(xd_ref[...], wr_ref[...],
                                (((1,), (1,)), ((), ())),
                                preferred_element_type=jnp.float32)
    acc = acc + bl_ref[...]
    if relu:
        acc = jnp.maximum(acc, 0.0)
    o_ref[...] = acc


def _make_combine(relu):
    return pl.pallas_call(
        functools.partial(_combine_body, relu),
        out_shape=jax.ShapeDtypeStruct((N, D), jnp.float32),
        grid=(N // BLK,),
        in_specs=[
            pl.BlockSpec((BLK, D), lambda i: (i, 0)),
            pl.BlockSpec((BLK, 1), lambda i: (i, 0)),
            pl.BlockSpec((BLK, D), lambda i: (i, 0)),
            pl.BlockSpec((D, D), lambda i: (0, 0)),
            pl.BlockSpec((1, D), lambda i: (0, 0)),
            pl.BlockSpec((D, D), lambda i: (0, 0)),
        ],
        out_specs=pl.BlockSpec((BLK, D), lambda i: (i, 0)),
    )


_combine_relu = _make_combine(True)
_combine_lin = _make_combine(False)


def kernel(x_paper, x_author, edge_index_cites, edge_index_rev_writes,
           Wl1c, bl1c, Wr1c, Wl1r, bl1r, Wr1r,
           Wl2c, bl2c, Wr2c, Wl2r, bl2r, Wr2r):
    src_c = edge_index_cites[0].astype(jnp.int32)
    dst_c = edge_index_cites[1].astype(jnp.int32)
    src_r = edge_index_rev_writes[0].astype(jnp.int32)
    dst_r = edge_index_rev_writes[1].astype(jnp.int32)

    sum_c1, sum_r1, cnt_c, cnt_r = _agg(x_paper, src_c, dst_c, src_r, dst_r)
    cnt_c = cnt_c[:, None]
    cnt_r = cnt_r[:, None]
    p1 = _combine_relu(sum_c1, cnt_c, x_paper,
                       Wl1c, bl1c.reshape(1, D), Wr1c)
    a1 = _combine_relu(sum_r1, cnt_r, x_author,
                       Wl1r, bl1r.reshape(1, D), Wr1r)

    sum_c2, sum_r2 = _agg_nc(p1, src_c, dst_c, src_r, dst_r)
    p2 = _combine_lin(sum_c2, cnt_c, p1, Wl2c, bl2c.reshape(1, D), Wr2c)
    a2 = _combine_lin(sum_r2, cnt_r, a1, Wl2r, bl2r.reshape(1, D), Wr2r)
    return (p2, a2)


# speedup vs baseline: 10.8815x; 1.0198x over previous
import functools

import jax
import jax.numpy as jnp
from jax import lax
from jax.experimental import pallas as pl
from jax.experimental.pallas import tpu as pltpu
from jax.experimental.pallas import tpu_sc as plsc

N = 10000
D = 128
E = 320000
NS = 16
CHUNK = 64
KPB = 8
NB = 4
EPT = E // NS
NFULL = EPT // CHUNK
NBLK = NFULL // KPB
TAIL = EPT - NFULL * CHUNK
RPT = 624
RTAIL = N - NS * RPT
_ZSIZES = (64, 64, 64, 64, 64, 64, 64, 64, 64, 48)


def _fill2d(ref, val):
    rows, cols = ref.shape

    def body(r, carry):
        for k in range(cols // 16):
            ref[r, pl.ds(k * 16, 16)] = jnp.full((16,), val, ref.dtype)
        return carry

    lax.fori_loop(0, rows, body, 0)


def _fill1d(ref, val):
    n, = ref.shape
    for k in range(n // 16):
        ref[pl.ds(k * 16, 16)] = jnp.full((16,), val, ref.dtype)


def _agg_body(with_counts, x_hbm, src_c_hbm, dst_c_hbm, src_r_hbm, dst_r_hbm,
              *args):
    if with_counts:
        sum_c_hbm, sum_r_hbm, cnt_c_hbm, cnt_r_hbm = args[:4]
        sc = args[4:]
    else:
        sum_c_hbm, sum_r_hbm = args[:2]
        sc = args[2:]
    idxs_v = sc[0:KPB]
    didxs_v = sc[KPB:2 * KPB]
    rows = sc[2 * KPB:2 * KPB + NB]
    gsem = sc[2 * KPB + NB:2 * KPB + 2 * NB]
    ssem = sc[2 * KPB + 2 * NB:2 * KPB + 3 * NB]
    (acc, cacc, idx_t, didx_t, z1_v, ones_v, cbuf_v, isem, csem) = \
        sc[2 * KPB + 3 * NB:]
    rows_v = rows[0]
    c = lax.axis_index("c")
    s = lax.axis_index("s")

    _fill2d(rows_v, 0.0)
    _fill1d(z1_v, 0.0)
    _fill1d(ones_v, 1.0)

    row0 = s * RPT
    zcps = []
    off = 0
    for n in _ZSIZES:
        zcps.append(pltpu.async_copy(rows_v.at[pl.ds(0, n)],
                                     acc.at[pl.ds(row0 + off, n)], isem))
        if with_counts:
            zcps.append(pltpu.async_copy(z1_v.at[pl.ds(0, n)],
                                         cacc.at[pl.ds(row0 + off, n)], csem))
        off += n
    for cp in zcps:
        cp.wait()

    @pl.when(s == 0)
    def _():
        pltpu.sync_copy(rows_v.at[pl.ds(0, RTAIL)],
                        acc.at[pl.ds(NS * RPT, RTAIL)])
        if with_counts:
            pltpu.sync_copy(z1_v.at[pl.ds(0, RTAIL)],
                            cacc.at[pl.ds(NS * RPT, RTAIL)])

    plsc.subcore_barrier()

    def run(src_hbm, dst_hbm):
        base0 = s * EPT

        def block(i, carry):
            base = base0 + i * (KPB * CHUNK)
            cps = []
            for k in range(KPB):
                cps.append(pltpu.async_copy(
                    src_hbm.at[pl.ds(base + k * CHUNK, CHUNK)], idxs_v[k], isem))
                cps.append(pltpu.async_copy(
                    dst_hbm.at[pl.ds(base + k * CHUNK, CHUNK)], didxs_v[k], isem))
            for cp in cps:
                cp.wait()
            g = [None] * KPB
            scp = [None] * KPB
            for k in range(NB - 1):
                g[k] = pltpu.async_copy(x_hbm.at[idxs_v[k]], rows[k % NB],
                                        gsem[k % NB])
            ccp = []
            for k in range(KPB):
                j = k + NB - 1
                if j < KPB:
                    if j - NB >= 0:
                        scp[j - NB].wait()
                    g[j] = pltpu.async_copy(x_hbm.at[idxs_v[j]], rows[j % NB],
                                            gsem[j % NB])
                g[k].wait()
                scp[k] = pltpu.async_copy(rows[k % NB], acc.at[didxs_v[k]],
                                          ssem[k % NB], add=True)
                if with_counts:
                    ccp.append(pltpu.async_copy(ones_v, cacc.at[didxs_v[k]],
                                                csem, add=True))
            for k in range(KPB - NB, KPB):
                scp[k].wait()
            for cp in ccp:
                cp.wait()
            return carry

        lax.fori_loop(0, NBLK, block, 0)

        base = base0 + NFULL * CHUNK
        pltpu.sync_copy(src_hbm.at[pl.ds(base, TAIL)], idx_t)
        pltpu.sync_copy(dst_hbm.at[pl.ds(base, TAIL)], didx_t)
        pltpu.async_copy(x_hbm.at[idx_t], rows[0].at[pl.ds(0, TAIL)],
                         gsem[0]).wait()
        pltpu.sync_copy(rows[0].at[pl.ds(0, TAIL)], acc.at[didx_t], add=True)
        if with_counts:
            pltpu.sync_copy(ones_v.at[pl.ds(0, TAIL)], cacc.at[didx_t],
                            add=True)

    @pl.when(c == 0)
    def _():
        run(src_c_hbm, dst_c_hbm)

    @pl.when(c == 1)
    def _():
        run(src_r_hbm, dst_r_hbm)

    plsc.subcore_barrier()

    def writeback(sum_hbm, cnt_hbm):
        wcps = []
        o = 0
        for n in _ZSIZES:
            r = row0 + o
            wcps.append(pltpu.async_copy(acc.at[pl.ds(r, n)],
                                         sum_hbm.at[pl.ds(r, n)], isem))
            o += n
        for cp in wcps:
            cp.wait()
        if with_counts:
            o = 0
            for n in _ZSIZES:
                r = row0 + o
                pltpu.sync_copy(cacc.at[pl.ds(r, n)], cbuf_v.at[pl.ds(0, n)])
                pltpu.sync_copy(cbuf_v.at[pl.ds(0, n)], cnt_hbm.at[pl.ds(r, n)])
                o += n

        @pl.when(s == 0)
        def _():
            r = NS * RPT
            pltpu.sync_copy(acc.at[pl.ds(r, RTAIL)], sum_hbm.at[pl.ds(r, RTAIL)])
            if with_counts:
                pltpu.sync_copy(cacc.at[pl.ds(r, RTAIL)],
                                cbuf_v.at[pl.ds(0, RTAIL)])
                pltpu.sync_copy(cbuf_v.at[pl.ds(0, RTAIL)],
                                cnt_hbm.at[pl.ds(r, RTAIL)])

    @pl.when(c == 0)
    def _():
        writeback(sum_c_hbm, cnt_c_hbm if with_counts else None)

    @pl.when(c == 1)
    def _():
        writeback(sum_r_hbm, cnt_r_hbm if with_counts else None)


def _make_agg(with_counts):
    outs = (
        jax.ShapeDtypeStruct((N, D), jnp.float32),
        jax.ShapeDtypeStruct((N, D), jnp.float32),
    )
    if with_counts:
        outs = outs + (
            jax.ShapeDtypeStruct((N,), jnp.float32),
            jax.ShapeDtypeStruct((N,), jnp.float32),
        )
    return pl.kernel(
        functools.partial(_agg_body, with_counts),
        out_type=outs,
        mesh=plsc.VectorSubcoreMesh(core_axis_name="c", subcore_axis_name="s"),
        scratch_types=(
            tuple(pltpu.VMEM((CHUNK,), jnp.int32) for _ in range(KPB))
            + tuple(pltpu.VMEM((CHUNK,), jnp.int32) for _ in range(KPB))
            + tuple(pltpu.VMEM((CHUNK, D), jnp.float32) for _ in range(NB))
            + tuple(pltpu.SemaphoreType.DMA for _ in range(NB))
            + tuple(pltpu.SemaphoreType.DMA for _ in range(NB))
            + (
                pltpu.VMEM_SHARED((N, D), jnp.float32),
                pltpu.VMEM_SHARED((N,), jnp.float32),
                pltpu.VMEM((TAIL,), jnp.int32),
                pltpu.VMEM((TAIL,), jnp.int32),
                pltpu.VMEM((CHUNK,), jnp.float32),
                pltpu.VMEM((CHUNK,), jnp.float32),
                pltpu.VMEM((CHUNK,), jnp.float32),
                pltpu.SemaphoreType.DMA,
                pltpu.SemaphoreType.DMA,
            )
        ),
    )


_agg = _make_agg(True)
_agg_nc = _make_agg(False)


BLK = 2000


def _combine_body(relu, sum_ref, cnt_ref, xd_ref, wl_ref, bl_ref, wr_ref,
                  o_ref):
    cnt = jnp.maximum(cnt_ref[...], 1.0)
    mean = sum_ref[...] / cnt
    acc = lax.dot_general(mean, wl_ref[...], (((1,), (1,)), ((), ())),
                          preferred_element_type=jnp.float32)
    acc = acc + lax.dot_general(xd_ref[...], wr_ref[...],
                                (((1,), (1,)), ((), ())),
                                preferred_element_type=jnp.float32)
    acc = acc + bl_ref[...]
    if relu:
        acc = jnp.maximum(acc, 0.0)
    o_ref[...] = acc


def _make_combine(relu):
    return pl.pallas_call(
        functools.partial(_combine_body, relu),
        out_shape=jax.ShapeDtypeStruct((N, D), jnp.float32),
        grid=(N // BLK,),
        in_specs=[
            pl.BlockSpec((BLK, D), lambda i: (i, 0)),
            pl.BlockSpec((BLK, 1), lambda i: (i, 0)),
            pl.BlockSpec((BLK, D), lambda i: (i, 0)),
            pl.BlockSpec((D, D), lambda i: (0, 0)),
            pl.BlockSpec((1, D), lambda i: (0, 0)),
            pl.BlockSpec((D, D), lambda i: (0, 0)),
        ],
        out_specs=pl.BlockSpec((BLK, D), lambda i: (i, 0)),
    )


_combine_relu = _make_combine(True)
_combine_lin = _make_combine(False)


def kernel(x_paper, x_author, edge_index_cites, edge_index_rev_writes,
           Wl1c, bl1c, Wr1c, Wl1r, bl1r, Wr1r,
           Wl2c, bl2c, Wr2c, Wl2r, bl2r, Wr2r):
    src_c = edge_index_cites[0].astype(jnp.int32)
    dst_c = edge_index_cites[1].astype(jnp.int32)
    src_r = edge_index_rev_writes[0].astype(jnp.int32)
    dst_r = edge_index_rev_writes[1].astype(jnp.int32)

    sum_c1, sum_r1, cnt_c, cnt_r = _agg(x_paper, src_c, dst_c, src_r, dst_r)
    cnt_c = cnt_c[:, None]
    cnt_r = cnt_r[:, None]
    p1 = _combine_relu(sum_c1, cnt_c, x_paper,
                       Wl1c, bl1c.reshape(1, D), Wr1c)
    a1 = _combine_relu(sum_r1, cnt_r, x_author,
                       Wl1r, bl1r.reshape(1, D), Wr1r)

    sum_c2, sum_r2 = _agg_nc(p1, src_c, dst_c, src_r, dst_r)
    p2 = _combine_lin(sum_c2, cnt_c, p1, Wl2c, bl2c.reshape(1, D), Wr2c)
    a2 = _combine_lin(sum_r2, cnt_r, a1, Wl2r, bl2r.reshape(1, D), Wr2r)
    return (p2, a2)


# speedup vs baseline: 12.0703x; 1.1093x over previous
import functools

import jax
import jax.numpy as jnp
from jax import lax
from jax.experimental import pallas as pl
from jax.experimental.pallas import tpu as pltpu
from jax.experimental.pallas import tpu_sc as plsc

N = 10000
D = 128
E = 320000
NS = 16
CHUNK = 64
KPB = 12
NB = 4
EPT = E // NS
NFULL = EPT // CHUNK
NBLK = NFULL // KPB
TAIL = EPT - NFULL * CHUNK
RPT = 624
RTAIL = N - NS * RPT
_ZSIZES = (64, 64, 64, 64, 64, 64, 64, 64, 64, 48)


def _fill2d(ref, val):
    rows, cols = ref.shape

    def body(r, carry):
        for k in range(cols // 16):
            ref[r, pl.ds(k * 16, 16)] = jnp.full((16,), val, ref.dtype)
        return carry

    lax.fori_loop(0, rows, body, 0)


def _fill1d(ref, val):
    n, = ref.shape
    for k in range(n // 16):
        ref[pl.ds(k * 16, 16)] = jnp.full((16,), val, ref.dtype)


def _agg_body(with_counts, x_hbm, src_c_hbm, dst_c_hbm, src_r_hbm, dst_r_hbm,
              *args):
    if with_counts:
        sum_c_hbm, sum_r_hbm, cnt_c_hbm, cnt_r_hbm = args[:4]
        sc = args[4:]
    else:
        sum_c_hbm, sum_r_hbm = args[:2]
        sc = args[2:]
    idxs_v = sc[0:KPB]
    didxs_v = sc[KPB:2 * KPB]
    rows = sc[2 * KPB:2 * KPB + NB]
    gsem = sc[2 * KPB + NB:2 * KPB + 2 * NB]
    ssem = sc[2 * KPB + 2 * NB:2 * KPB + 3 * NB]
    (acc, cacc, idx_t, didx_t, z1_v, ones_v, cbuf_v, isem, csem) = \
        sc[2 * KPB + 3 * NB:]
    rows_v = rows[0]
    c = lax.axis_index("c")
    s = lax.axis_index("s")

    _fill2d(rows_v, 0.0)
    _fill1d(z1_v, 0.0)
    _fill1d(ones_v, 1.0)

    row0 = s * RPT
    zcps = []
    off = 0
    for n in _ZSIZES:
        zcps.append(pltpu.async_copy(rows_v.at[pl.ds(0, n)],
                                     acc.at[pl.ds(row0 + off, n)], isem))
        if with_counts:
            zcps.append(pltpu.async_copy(z1_v.at[pl.ds(0, n)],
                                         cacc.at[pl.ds(row0 + off, n)], csem))
        off += n
    for cp in zcps:
        cp.wait()

    @pl.when(s == 0)
    def _():
        pltpu.sync_copy(rows_v.at[pl.ds(0, RTAIL)],
                        acc.at[pl.ds(NS * RPT, RTAIL)])
        if with_counts:
            pltpu.sync_copy(z1_v.at[pl.ds(0, RTAIL)],
                            cacc.at[pl.ds(NS * RPT, RTAIL)])

    plsc.subcore_barrier()

    def run(src_hbm, dst_hbm):
        base0 = s * EPT

        def block(i, carry):
            base = base0 + i * (KPB * CHUNK)
            cps = []
            for k in range(KPB):
                cps.append(pltpu.async_copy(
                    src_hbm.at[pl.ds(base + k * CHUNK, CHUNK)], idxs_v[k], isem))
                cps.append(pltpu.async_copy(
                    dst_hbm.at[pl.ds(base + k * CHUNK, CHUNK)], didxs_v[k], isem))
            for cp in cps:
                cp.wait()
            g = [None] * KPB
            scp = [None] * KPB
            for k in range(NB - 1):
                g[k] = pltpu.async_copy(x_hbm.at[idxs_v[k]], rows[k % NB],
                                        gsem[k % NB])
            ccp = []
            for k in range(KPB):
                j = k + NB - 1
                if j < KPB:
                    if j - NB >= 0:
                        scp[j - NB].wait()
                    g[j] = pltpu.async_copy(x_hbm.at[idxs_v[j]], rows[j % NB],
                                            gsem[j % NB])
                g[k].wait()
                scp[k] = pltpu.async_copy(rows[k % NB], acc.at[didxs_v[k]],
                                          ssem[k % NB], add=True)
                if with_counts:
                    ccp.append(pltpu.async_copy(ones_v, cacc.at[didxs_v[k]],
                                                csem, add=True))
            for k in range(KPB - NB, KPB):
                scp[k].wait()
            for cp in ccp:
                cp.wait()
            return carry

        lax.fori_loop(0, NBLK, block, 0)

        base = base0 + NFULL * CHUNK
        pltpu.sync_copy(src_hbm.at[pl.ds(base, TAIL)], idx_t)
        pltpu.sync_copy(dst_hbm.at[pl.ds(base, TAIL)], didx_t)
        pltpu.async_copy(x_hbm.at[idx_t], rows[0].at[pl.ds(0, TAIL)],
                         gsem[0]).wait()
        pltpu.sync_copy(rows[0].at[pl.ds(0, TAIL)], acc.at[didx_t], add=True)
        if with_counts:
            pltpu.sync_copy(ones_v.at[pl.ds(0, TAIL)], cacc.at[didx_t],
                            add=True)

    @pl.when(c == 0)
    def _():
        run(src_c_hbm, dst_c_hbm)

    @pl.when(c == 1)
    def _():
        run(src_r_hbm, dst_r_hbm)

    plsc.subcore_barrier()

    def writeback(sum_hbm, cnt_hbm):
        wcps = []
        o = 0
        for n in _ZSIZES:
            r = row0 + o
            wcps.append(pltpu.async_copy(acc.at[pl.ds(r, n)],
                                         sum_hbm.at[pl.ds(r, n)], isem))
            o += n
        for cp in wcps:
            cp.wait()
        if with_counts:
            o = 0
            for n in _ZSIZES:
                r = row0 + o
                pltpu.sync_copy(cacc.at[pl.ds(r, n)], cbuf_v.at[pl.ds(0, n)])
                pltpu.sync_copy(cbuf_v.at[pl.ds(0, n)], cnt_hbm.at[pl.ds(r, n)])
                o += n

        @pl.when(s == 0)
        def _():
            r = NS * RPT
            pltpu.sync_copy(acc.at[pl.ds(r, RTAIL)], sum_hbm.at[pl.ds(r, RTAIL)])
            if with_counts:
                pltpu.sync_copy(cacc.at[pl.ds(r, RTAIL)],
                                cbuf_v.at[pl.ds(0, RTAIL)])
                pltpu.sync_copy(cbuf_v.at[pl.ds(0, RTAIL)],
                                cnt_hbm.at[pl.ds(r, RTAIL)])

    @pl.when(c == 0)
    def _():
        writeback(sum_c_hbm, cnt_c_hbm if with_counts else None)

    @pl.when(c == 1)
    def _():
        writeback(sum_r_hbm, cnt_r_hbm if with_counts else None)


def _make_agg(with_counts):
    outs = (
        jax.ShapeDtypeStruct((N, D), jnp.float32),
        jax.ShapeDtypeStruct((N, D), jnp.float32),
    )
    if with_counts:
        outs = outs + (
            jax.ShapeDtypeStruct((N,), jnp.float32),
            jax.ShapeDtypeStruct((N,), jnp.float32),
        )
    return pl.kernel(
        functools.partial(_agg_body, with_counts),
        out_type=outs,
        mesh=plsc.VectorSubcoreMesh(core_axis_name="c", subcore_axis_name="s"),
        scratch_types=(
            tuple(pltpu.VMEM((CHUNK,), jnp.int32) for _ in range(KPB))
            + tuple(pltpu.VMEM((CHUNK,), jnp.int32) for _ in range(KPB))
            + tuple(pltpu.VMEM((CHUNK, D), jnp.float32) for _ in range(NB))
            + tuple(pltpu.SemaphoreType.DMA for _ in range(NB))
            + tuple(pltpu.SemaphoreType.DMA for _ in range(NB))
            + (
                pltpu.VMEM_SHARED((N, D), jnp.float32),
                pltpu.VMEM_SHARED((N,), jnp.float32),
                pltpu.VMEM((TAIL,), jnp.int32),
                pltpu.VMEM((TAIL,), jnp.int32),
                pltpu.VMEM((CHUNK,), jnp.float32),
                pltpu.VMEM((CHUNK,), jnp.float32),
                pltpu.VMEM((CHUNK,), jnp.float32),
                pltpu.SemaphoreType.DMA,
                pltpu.SemaphoreType.DMA,
            )
        ),
    )


_agg = _make_agg(True)
_agg_nc = _make_agg(False)


BLK = 2000


def _combine_body(relu, sum_ref, cnt_ref, xd_ref, wl_ref, bl_ref, wr_ref,
                  o_ref):
    cnt = jnp.maximum(cnt_ref[...], 1.0)
    mean = sum_ref[...] / cnt
    acc = lax.dot_general(mean, wl_ref[...], (((1,), (1,)), ((), ())),
                          preferred_element_type=jnp.float32)
    acc = acc + lax.dot_general(xd_ref[...], wr_ref[...],
                                (((1,), (1,)), ((), ())),
                                preferred_element_type=jnp.float32)
    acc = acc + bl_ref[...]
    if relu:
        acc = jnp.maximum(acc, 0.0)
    o_ref[...] = acc


def _make_combine(relu):
    return pl.pallas_call(
        functools.partial(_combine_body, relu),
        out_shape=jax.ShapeDtypeStruct((N, D), jnp.float32),
        grid=(N // BLK,),
        in_specs=[
            pl.BlockSpec((BLK, D), lambda i: (i, 0)),
            pl.BlockSpec((BLK, 1), lambda i: (i, 0)),
            pl.BlockSpec((BLK, D), lambda i: (i, 0)),
            pl.BlockSpec((D, D), lambda i: (0, 0)),
            pl.BlockSpec((1, D), lambda i: (0, 0)),
            pl.BlockSpec((D, D), lambda i: (0, 0)),
        ],
        out_specs=pl.BlockSpec((BLK, D), lambda i: (i, 0)),
    )


_combine_relu = _make_combine(True)
_combine_lin = _make_combine(False)


def kernel(x_paper, x_author, edge_index_cites, edge_index_rev_writes,
           Wl1c, bl1c, Wr1c, Wl1r, bl1r, Wr1r,
           Wl2c, bl2c, Wr2c, Wl2r, bl2r, Wr2r):
    src_c = edge_index_cites[0].astype(jnp.int32)
    dst_c = edge_index_cites[1].astype(jnp.int32)
    src_r = edge_index_rev_writes[0].astype(jnp.int32)
    dst_r = edge_index_rev_writes[1].astype(jnp.int32)

    sum_c1, sum_r1, cnt_c, cnt_r = _agg(x_paper, src_c, dst_c, src_r, dst_r)
    cnt_c = cnt_c[:, None]
    cnt_r = cnt_r[:, None]
    p1 = _combine_relu(sum_c1, cnt_c, x_paper,
                       Wl1c, bl1c.reshape(1, D), Wr1c)
    a1 = _combine_relu(sum_r1, cnt_r, x_author,
                       Wl1r, bl1r.reshape(1, D), Wr1r)

    sum_c2, sum_r2 = _agg_nc(p1, src_c, dst_c, src_r, dst_r)
    p2 = _combine_lin(sum_c2, cnt_c, p1, Wl2c, bl2c.reshape(1, D), Wr2c)
    a2 = _combine_lin(sum_r2, cnt_r, a1, Wl2r, bl2r.reshape(1, D), Wr2r)
    return (p2, a2)


# speedup vs baseline: 13.4309x; 1.1127x over previous
import functools

import jax
import jax.numpy as jnp
from jax import lax
from jax.experimental import pallas as pl
from jax.experimental.pallas import tpu as pltpu
from jax.experimental.pallas import tpu_sc as plsc

N = 10000
D = 128
E = 320000
NS = 16
CHUNK = 64
KPB = 24
NB = 4
EPT = E // NS
NFULL = EPT // CHUNK
NBLK = NFULL // KPB
TAIL = EPT - NFULL * CHUNK
RPT = 624
RTAIL = N - NS * RPT
_ZSIZES = (64, 64, 64, 64, 64, 64, 64, 64, 64, 48)


def _fill2d(ref, val):
    rows, cols = ref.shape

    def body(r, carry):
        for k in range(cols // 16):
            ref[r, pl.ds(k * 16, 16)] = jnp.full((16,), val, ref.dtype)
        return carry

    lax.fori_loop(0, rows, body, 0)


def _fill1d(ref, val):
    n, = ref.shape
    for k in range(n // 16):
        ref[pl.ds(k * 16, 16)] = jnp.full((16,), val, ref.dtype)


def _agg_body(with_counts, x_hbm, src_c_hbm, dst_c_hbm, src_r_hbm, dst_r_hbm,
              *args):
    if with_counts:
        sum_c_hbm, sum_r_hbm, cnt_c_hbm, cnt_r_hbm = args[:4]
        sc = args[4:]
    else:
        sum_c_hbm, sum_r_hbm = args[:2]
        sc = args[2:]
    idxs_v = sc[0:KPB]
    didxs_v = sc[KPB:2 * KPB]
    rows = sc[2 * KPB:2 * KPB + NB]
    gsem = sc[2 * KPB + NB:2 * KPB + 2 * NB]
    ssem = sc[2 * KPB + 2 * NB:2 * KPB + 3 * NB]
    (acc, cacc, idx_t, didx_t, z1_v, ones_v, cbuf_v, isem, csem) = \
        sc[2 * KPB + 3 * NB:]
    rows_v = rows[0]
    c = lax.axis_index("c")
    s = lax.axis_index("s")

    _fill2d(rows_v, 0.0)
    _fill1d(z1_v, 0.0)
    _fill1d(ones_v, 1.0)

    row0 = s * RPT
    zcps = []
    off = 0
    for n in _ZSIZES:
        zcps.append(pltpu.async_copy(rows_v.at[pl.ds(0, n)],
                                     acc.at[pl.ds(row0 + off, n)], isem))
        if with_counts:
            zcps.append(pltpu.async_copy(z1_v.at[pl.ds(0, n)],
                                         cacc.at[pl.ds(row0 + off, n)], csem))
        off += n
    for cp in zcps:
        cp.wait()

    @pl.when(s == 0)
    def _():
        pltpu.sync_copy(rows_v.at[pl.ds(0, RTAIL)],
                        acc.at[pl.ds(NS * RPT, RTAIL)])
        if with_counts:
            pltpu.sync_copy(z1_v.at[pl.ds(0, RTAIL)],
                            cacc.at[pl.ds(NS * RPT, RTAIL)])

    plsc.subcore_barrier()

    def run(src_hbm, dst_hbm):
        base0 = s * EPT

        def block(i, carry):
            base = base0 + i * (KPB * CHUNK)
            cps = []
            for k in range(KPB):
                cps.append(pltpu.async_copy(
                    src_hbm.at[pl.ds(base + k * CHUNK, CHUNK)], idxs_v[k], isem))
                cps.append(pltpu.async_copy(
                    dst_hbm.at[pl.ds(base + k * CHUNK, CHUNK)], didxs_v[k], isem))
            for cp in cps:
                cp.wait()
            g = [None] * KPB
            scp = [None] * KPB
            for k in range(NB - 1):
                g[k] = pltpu.async_copy(x_hbm.at[idxs_v[k]], rows[k % NB],
                                        gsem[k % NB])
            ccp = []
            for k in range(KPB):
                j = k + NB - 1
                if j < KPB:
                    if j - NB >= 0:
                        scp[j - NB].wait()
                    g[j] = pltpu.async_copy(x_hbm.at[idxs_v[j]], rows[j % NB],
                                            gsem[j % NB])
                g[k].wait()
                scp[k] = pltpu.async_copy(rows[k % NB], acc.at[didxs_v[k]],
                                          ssem[k % NB], add=True)
                if with_counts:
                    ccp.append(pltpu.async_copy(ones_v, cacc.at[didxs_v[k]],
                                                csem, add=True))
            for k in range(KPB - NB, KPB):
                scp[k].wait()
            for cp in ccp:
                cp.wait()
            return carry

        lax.fori_loop(0, NBLK, block, 0)

        base = base0 + NFULL * CHUNK
        pltpu.sync_copy(src_hbm.at[pl.ds(base, TAIL)], idx_t)
        pltpu.sync_copy(dst_hbm.at[pl.ds(base, TAIL)], didx_t)
        pltpu.async_copy(x_hbm.at[idx_t], rows[0].at[pl.ds(0, TAIL)],
                         gsem[0]).wait()
        pltpu.sync_copy(rows[0].at[pl.ds(0, TAIL)], acc.at[didx_t], add=True)
        if with_counts:
            pltpu.sync_copy(ones_v.at[pl.ds(0, TAIL)], cacc.at[didx_t],
                            add=True)

    @pl.when(c == 0)
    def _():
        run(src_c_hbm, dst_c_hbm)

    @pl.when(c == 1)
    def _():
        run(src_r_hbm, dst_r_hbm)

    plsc.subcore_barrier()

    def writeback(sum_hbm, cnt_hbm):
        wcps = []
        o = 0
        for n in _ZSIZES:
            r = row0 + o
            wcps.append(pltpu.async_copy(acc.at[pl.ds(r, n)],
                                         sum_hbm.at[pl.ds(r, n)], isem))
            o += n
        for cp in wcps:
            cp.wait()
        if with_counts:
            o = 0
            for n in _ZSIZES:
                r = row0 + o
                pltpu.sync_copy(cacc.at[pl.ds(r, n)], cbuf_v.at[pl.ds(0, n)])
                pltpu.sync_copy(cbuf_v.at[pl.ds(0, n)], cnt_hbm.at[pl.ds(r, n)])
                o += n

        @pl.when(s == 0)
        def _():
            r = NS * RPT
            pltpu.sync_copy(acc.at[pl.ds(r, RTAIL)], sum_hbm.at[pl.ds(r, RTAIL)])
            if with_counts:
                pltpu.sync_copy(cacc.at[pl.ds(r, RTAIL)],
                                cbuf_v.at[pl.ds(0, RTAIL)])
                pltpu.sync_copy(cbuf_v.at[pl.ds(0, RTAIL)],
                                cnt_hbm.at[pl.ds(r, RTAIL)])

    @pl.when(c == 0)
    def _():
        writeback(sum_c_hbm, cnt_c_hbm if with_counts else None)

    @pl.when(c == 1)
    def _():
        writeback(sum_r_hbm, cnt_r_hbm if with_counts else None)


def _make_agg(with_counts):
    outs = (
        jax.ShapeDtypeStruct((N, D), jnp.float32),
        jax.ShapeDtypeStruct((N, D), jnp.float32),
    )
    if with_counts:
        outs = outs + (
            jax.ShapeDtypeStruct((N,), jnp.float32),
            jax.ShapeDtypeStruct((N,), jnp.float32),
        )
    return pl.kernel(
        functools.partial(_agg_body, with_counts),
        out_type=outs,
        mesh=plsc.VectorSubcoreMesh(core_axis_name="c", subcore_axis_name="s"),
        scratch_types=(
            tuple(pltpu.VMEM((CHUNK,), jnp.int32) for _ in range(KPB))
            + tuple(pltpu.VMEM((CHUNK,), jnp.int32) for _ in range(KPB))
            + tuple(pltpu.VMEM((CHUNK, D), jnp.float32) for _ in range(NB))
            + tuple(pltpu.SemaphoreType.DMA for _ in range(NB))
            + tuple(pltpu.SemaphoreType.DMA for _ in range(NB))
            + (
                pltpu.VMEM_SHARED((N, D), jnp.float32),
                pltpu.VMEM_SHARED((N,), jnp.float32),
                pltpu.VMEM((TAIL,), jnp.int32),
                pltpu.VMEM((TAIL,), jnp.int32),
                pltpu.VMEM((CHUNK,), jnp.float32),
                pltpu.VMEM((CHUNK,), jnp.float32),
                pltpu.VMEM((CHUNK,), jnp.float32),
                pltpu.SemaphoreType.DMA,
                pltpu.SemaphoreType.DMA,
            )
        ),
    )


_agg = _make_agg(True)
_agg_nc = _make_agg(False)


BLK = 2000


def _combine_body(relu, sum_ref, cnt_ref, xd_ref, wl_ref, bl_ref, wr_ref,
                  o_ref):
    cnt = jnp.maximum(cnt_ref[...], 1.0)
    mean = sum_ref[...] / cnt
    acc = lax.dot_general(mean, wl_ref[...], (((1,), (1,)), ((), ())),
                          preferred_element_type=jnp.float32)
    acc = acc + lax.dot_general(xd_ref[...], wr_ref[...],
                                (((1,), (1,)), ((), ())),
                                preferred_element_type=jnp.float32)
    acc = acc + bl_ref[...]
    if relu:
        acc = jnp.maximum(acc, 0.0)
    o_ref[...] = acc


def _make_combine(relu):
    return pl.pallas_call(
        functools.partial(_combine_body, relu),
        out_shape=jax.ShapeDtypeStruct((N, D), jnp.float32),
        grid=(N // BLK,),
        in_specs=[
            pl.BlockSpec((BLK, D), lambda i: (i, 0)),
            pl.BlockSpec((BLK, 1), lambda i: (i, 0)),
            pl.BlockSpec((BLK, D), lambda i: (i, 0)),
            pl.BlockSpec((D, D), lambda i: (0, 0)),
            pl.BlockSpec((1, D), lambda i: (0, 0)),
            pl.BlockSpec((D, D), lambda i: (0, 0)),
        ],
        out_specs=pl.BlockSpec((BLK, D), lambda i: (i, 0)),
    )


_combine_relu = _make_combine(True)
_combine_lin = _make_combine(False)


def kernel(x_paper, x_author, edge_index_cites, edge_index_rev_writes,
           Wl1c, bl1c, Wr1c, Wl1r, bl1r, Wr1r,
           Wl2c, bl2c, Wr2c, Wl2r, bl2r, Wr2r):
    src_c = edge_index_cites[0].astype(jnp.int32)
    dst_c = edge_index_cites[1].astype(jnp.int32)
    src_r = edge_index_rev_writes[0].astype(jnp.int32)
    dst_r = edge_index_rev_writes[1].astype(jnp.int32)

    sum_c1, sum_r1, cnt_c, cnt_r = _agg(x_paper, src_c, dst_c, src_r, dst_r)
    cnt_c = cnt_c[:, None]
    cnt_r = cnt_r[:, None]
    p1 = _combine_relu(sum_c1, cnt_c, x_paper,
                       Wl1c, bl1c.reshape(1, D), Wr1c)
    a1 = _combine_relu(sum_r1, cnt_r, x_author,
                       Wl1r, bl1r.reshape(1, D), Wr1r)

    sum_c2, sum_r2 = _agg_nc(p1, src_c, dst_c, src_r, dst_r)
    p2 = _combine_lin(sum_c2, cnt_c, p1, Wl2c, bl2c.reshape(1, D), Wr2c)
    a2 = _combine_lin(sum_r2, cnt_r, a1, Wl2r, bl2r.reshape(1, D), Wr2r)
    return (p2, a2)


# speedup vs baseline: 13.9281x; 1.0370x over previous
import functools

import jax
import jax.numpy as jnp
from jax import lax
from jax.experimental import pallas as pl
from jax.experimental.pallas import tpu as pltpu
from jax.experimental.pallas import tpu_sc as plsc

N = 10000
D = 128
E = 320000
NS = 16
CHUNK = 64
KPB = 39
NB = 4
EPT = E // NS
NFULL = EPT // CHUNK
NBLK = NFULL // KPB
TAIL = EPT - NFULL * CHUNK
RPT = 624
RTAIL = N - NS * RPT
_ZSIZES = (64, 64, 64, 64, 64, 64, 64, 64, 64, 48)


def _fill2d(ref, val):
    rows, cols = ref.shape

    def body(r, carry):
        for k in range(cols // 16):
            ref[r, pl.ds(k * 16, 16)] = jnp.full((16,), val, ref.dtype)
        return carry

    lax.fori_loop(0, rows, body, 0)


def _fill1d(ref, val):
    n, = ref.shape
    for k in range(n // 16):
        ref[pl.ds(k * 16, 16)] = jnp.full((16,), val, ref.dtype)


def _agg_body(with_counts, x_hbm, src_c_hbm, dst_c_hbm, src_r_hbm, dst_r_hbm,
              *args):
    if with_counts:
        sum_c_hbm, sum_r_hbm, cnt_c_hbm, cnt_r_hbm = args[:4]
        sc = args[4:]
    else:
        sum_c_hbm, sum_r_hbm = args[:2]
        sc = args[2:]
    idxs_v = sc[0:KPB]
    didxs_v = sc[KPB:2 * KPB]
    rows = sc[2 * KPB:2 * KPB + NB]
    gsem = sc[2 * KPB + NB:2 * KPB + 2 * NB]
    ssem = sc[2 * KPB + 2 * NB:2 * KPB + 3 * NB]
    (acc, cacc, idx_t, didx_t, z1_v, ones_v, cbuf_v, isem, csem) = \
        sc[2 * KPB + 3 * NB:]
    rows_v = rows[0]
    c = lax.axis_index("c")
    s = lax.axis_index("s")

    _fill2d(rows_v, 0.0)
    _fill1d(z1_v, 0.0)
    _fill1d(ones_v, 1.0)

    row0 = s * RPT
    zcps = []
    off = 0
    for n in _ZSIZES:
        zcps.append(pltpu.async_copy(rows_v.at[pl.ds(0, n)],
                                     acc.at[pl.ds(row0 + off, n)], isem))
        if with_counts:
            zcps.append(pltpu.async_copy(z1_v.at[pl.ds(0, n)],
                                         cacc.at[pl.ds(row0 + off, n)], csem))
        off += n
    for cp in zcps:
        cp.wait()

    @pl.when(s == 0)
    def _():
        pltpu.sync_copy(rows_v.at[pl.ds(0, RTAIL)],
                        acc.at[pl.ds(NS * RPT, RTAIL)])
        if with_counts:
            pltpu.sync_copy(z1_v.at[pl.ds(0, RTAIL)],
                            cacc.at[pl.ds(NS * RPT, RTAIL)])

    plsc.subcore_barrier()

    def run(src_hbm, dst_hbm):
        base0 = s * EPT

        def block(i, carry):
            base = base0 + i * (KPB * CHUNK)
            cps = []
            for k in range(KPB):
                cps.append(pltpu.async_copy(
                    src_hbm.at[pl.ds(base + k * CHUNK, CHUNK)], idxs_v[k], isem))
                cps.append(pltpu.async_copy(
                    dst_hbm.at[pl.ds(base + k * CHUNK, CHUNK)], didxs_v[k], isem))
            for cp in cps:
                cp.wait()
            g = [None] * KPB
            scp = [None] * KPB
            for k in range(NB - 1):
                g[k] = pltpu.async_copy(x_hbm.at[idxs_v[k]], rows[k % NB],
                                        gsem[k % NB])
            ccp = []
            for k in range(KPB):
                j = k + NB - 1
                if j < KPB:
                    if j - NB >= 0:
                        scp[j - NB].wait()
                    g[j] = pltpu.async_copy(x_hbm.at[idxs_v[j]], rows[j % NB],
                                            gsem[j % NB])
                g[k].wait()
                scp[k] = pltpu.async_copy(rows[k % NB], acc.at[didxs_v[k]],
                                          ssem[k % NB], add=True)
                if with_counts:
                    ccp.append(pltpu.async_copy(ones_v, cacc.at[didxs_v[k]],
                                                csem, add=True))
            for k in range(KPB - NB, KPB):
                scp[k].wait()
            for cp in ccp:
                cp.wait()
            return carry

        lax.fori_loop(0, NBLK, block, 0)

        base = base0 + NFULL * CHUNK
        pltpu.sync_copy(src_hbm.at[pl.ds(base, TAIL)], idx_t)
        pltpu.sync_copy(dst_hbm.at[pl.ds(base, TAIL)], didx_t)
        pltpu.async_copy(x_hbm.at[idx_t], rows[0].at[pl.ds(0, TAIL)],
                         gsem[0]).wait()
        pltpu.sync_copy(rows[0].at[pl.ds(0, TAIL)], acc.at[didx_t], add=True)
        if with_counts:
            pltpu.sync_copy(ones_v.at[pl.ds(0, TAIL)], cacc.at[didx_t],
                            add=True)

    @pl.when(c == 0)
    def _():
        run(src_c_hbm, dst_c_hbm)

    @pl.when(c == 1)
    def _():
        run(src_r_hbm, dst_r_hbm)

    plsc.subcore_barrier()

    def writeback(sum_hbm, cnt_hbm):
        wcps = []
        o = 0
        for n in _ZSIZES:
            r = row0 + o
            wcps.append(pltpu.async_copy(acc.at[pl.ds(r, n)],
                                         sum_hbm.at[pl.ds(r, n)], isem))
            o += n
        for cp in wcps:
            cp.wait()
        if with_counts:
            o = 0
            for n in _ZSIZES:
                r = row0 + o
                pltpu.sync_copy(cacc.at[pl.ds(r, n)], cbuf_v.at[pl.ds(0, n)])
                pltpu.sync_copy(cbuf_v.at[pl.ds(0, n)], cnt_hbm.at[pl.ds(r, n)])
                o += n

        @pl.when(s == 0)
        def _():
            r = NS * RPT
            pltpu.sync_copy(acc.at[pl.ds(r, RTAIL)], sum_hbm.at[pl.ds(r, RTAIL)])
            if with_counts:
                pltpu.sync_copy(cacc.at[pl.ds(r, RTAIL)],
                                cbuf_v.at[pl.ds(0, RTAIL)])
                pltpu.sync_copy(cbuf_v.at[pl.ds(0, RTAIL)],
                                cnt_hbm.at[pl.ds(r, RTAIL)])

    @pl.when(c == 0)
    def _():
        writeback(sum_c_hbm, cnt_c_hbm if with_counts else None)

    @pl.when(c == 1)
    def _():
        writeback(sum_r_hbm, cnt_r_hbm if with_counts else None)


def _make_agg(with_counts):
    outs = (
        jax.ShapeDtypeStruct((N, D), jnp.float32),
        jax.ShapeDtypeStruct((N, D), jnp.float32),
    )
    if with_counts:
        outs = outs + (
            jax.ShapeDtypeStruct((N,), jnp.float32),
            jax.ShapeDtypeStruct((N,), jnp.float32),
        )
    return pl.kernel(
        functools.partial(_agg_body, with_counts),
        out_type=outs,
        mesh=plsc.VectorSubcoreMesh(core_axis_name="c", subcore_axis_name="s"),
        scratch_types=(
            tuple(pltpu.VMEM((CHUNK,), jnp.int32) for _ in range(KPB))
            + tuple(pltpu.VMEM((CHUNK,), jnp.int32) for _ in range(KPB))
            + tuple(pltpu.VMEM((CHUNK, D), jnp.float32) for _ in range(NB))
            + tuple(pltpu.SemaphoreType.DMA for _ in range(NB))
            + tuple(pltpu.SemaphoreType.DMA for _ in range(NB))
            + (
                pltpu.VMEM_SHARED((N, D), jnp.float32),
                pltpu.VMEM_SHARED((N,), jnp.float32),
                pltpu.VMEM((TAIL,), jnp.int32),
                pltpu.VMEM((TAIL,), jnp.int32),
                pltpu.VMEM((CHUNK,), jnp.float32),
                pltpu.VMEM((CHUNK,), jnp.float32),
                pltpu.VMEM((CHUNK,), jnp.float32),
                pltpu.SemaphoreType.DMA,
                pltpu.SemaphoreType.DMA,
            )
        ),
    )


_agg = _make_agg(True)
_agg_nc = _make_agg(False)


BLK = 2000


def _combine_body(relu, sum_ref, cnt_ref, xd_ref, wl_ref, bl_ref, wr_ref,
                  o_ref):
    cnt = jnp.maximum(cnt_ref[...], 1.0)
    mean = sum_ref[...] / cnt
    acc = lax.dot_general(mean, wl_ref[...], (((1,), (1,)), ((), ())),
                          preferred_element_type=jnp.float32)
    acc = acc + lax.dot_general(xd_ref[...], wr_ref[...],
                                (((1,), (1,)), ((), ())),
                                preferred_element_type=jnp.float32)
    acc = acc + bl_ref[...]
    if relu:
        acc = jnp.maximum(acc, 0.0)
    o_ref[...] = acc


def _make_combine(relu):
    return pl.pallas_call(
        functools.partial(_combine_body, relu),
        out_shape=jax.ShapeDtypeStruct((N, D), jnp.float32),
        grid=(N // BLK,),
        in_specs=[
            pl.BlockSpec((BLK, D), lambda i: (i, 0)),
            pl.BlockSpec((BLK, 1), lambda i: (i, 0)),
            pl.BlockSpec((BLK, D), lambda i: (i, 0)),
            pl.BlockSpec((D, D), lambda i: (0, 0)),
            pl.BlockSpec((1, D), lambda i: (0, 0)),
            pl.BlockSpec((D, D), lambda i: (0, 0)),
        ],
        out_specs=pl.BlockSpec((BLK, D), lambda i: (i, 0)),
    )


_combine_relu = _make_combine(True)
_combine_lin = _make_combine(False)


def kernel(x_paper, x_author, edge_index_cites, edge_index_rev_writes,
           Wl1c, bl1c, Wr1c, Wl1r, bl1r, Wr1r,
           Wl2c, bl2c, Wr2c, Wl2r, bl2r, Wr2r):
    src_c = edge_index_cites[0].astype(jnp.int32)
    dst_c = edge_index_cites[1].astype(jnp.int32)
    src_r = edge_index_rev_writes[0].astype(jnp.int32)
    dst_r = edge_index_rev_writes[1].astype(jnp.int32)

    sum_c1, sum_r1, cnt_c, cnt_r = _agg(x_paper, src_c, dst_c, src_r, dst_r)
    cnt_c = cnt_c[:, None]
    cnt_r = cnt_r[:, None]
    p1 = _combine_relu(sum_c1, cnt_c, x_paper,
                       Wl1c, bl1c.reshape(1, D), Wr1c)
    a1 = _combine_relu(sum_r1, cnt_r, x_author,
                       Wl1r, bl1r.reshape(1, D), Wr1r)

    sum_c2, sum_r2 = _agg_nc(p1, src_c, dst_c, src_r, dst_r)
    p2 = _combine_lin(sum_c2, cnt_c, p1, Wl2c, bl2c.reshape(1, D), Wr2c)
    a2 = _combine_lin(sum_r2, cnt_r, a1, Wl2r, bl2r.reshape(1, D), Wr2r)
    return (p2, a2)


# speedup vs baseline: 14.3038x; 1.0270x over previous
import functools

import jax
import jax.numpy as jnp
from jax import lax
from jax.experimental import pallas as pl
from jax.experimental.pallas import tpu as pltpu
from jax.experimental.pallas import tpu_sc as plsc

N = 10000
D = 128
E = 320000
NS = 16
CHUNK = 64
KPB = 52
NB = 4
EPT = E // NS
NFULL = EPT // CHUNK
NBLK = NFULL // KPB
TAIL = EPT - NFULL * CHUNK
RPT = 624
RTAIL = N - NS * RPT
_ZSIZES = (64, 64, 64, 64, 64, 64, 64, 64, 64, 48)


def _fill2d(ref, val):
    rows, cols = ref.shape

    def body(r, carry):
        for k in range(cols // 16):
            ref[r, pl.ds(k * 16, 16)] = jnp.full((16,), val, ref.dtype)
        return carry

    lax.fori_loop(0, rows, body, 0)


def _fill1d(ref, val):
    n, = ref.shape
    for k in range(n // 16):
        ref[pl.ds(k * 16, 16)] = jnp.full((16,), val, ref.dtype)


def _agg_body(with_counts, x_hbm, src_c_hbm, dst_c_hbm, src_r_hbm, dst_r_hbm,
              *args):
    if with_counts:
        sum_c_hbm, sum_r_hbm, cnt_c_hbm, cnt_r_hbm = args[:4]
        sc = args[4:]
    else:
        sum_c_hbm, sum_r_hbm = args[:2]
        sc = args[2:]
    idxs_v = sc[0:KPB]
    didxs_v = sc[KPB:2 * KPB]
    rows = sc[2 * KPB:2 * KPB + NB]
    gsem = sc[2 * KPB + NB:2 * KPB + 2 * NB]
    ssem = sc[2 * KPB + 2 * NB:2 * KPB + 3 * NB]
    (acc, cacc, idx_t, didx_t, z1_v, ones_v, cbuf_v, isem, csem) = \
        sc[2 * KPB + 3 * NB:]
    rows_v = rows[0]
    c = lax.axis_index("c")
    s = lax.axis_index("s")

    _fill2d(rows_v, 0.0)
    _fill1d(z1_v, 0.0)
    _fill1d(ones_v, 1.0)

    row0 = s * RPT
    zcps = []
    off = 0
    for n in _ZSIZES:
        zcps.append(pltpu.async_copy(rows_v.at[pl.ds(0, n)],
                                     acc.at[pl.ds(row0 + off, n)], isem))
        if with_counts:
            zcps.append(pltpu.async_copy(z1_v.at[pl.ds(0, n)],
                                         cacc.at[pl.ds(row0 + off, n)], csem))
        off += n
    for cp in zcps:
        cp.wait()

    @pl.when(s == 0)
    def _():
        pltpu.sync_copy(rows_v.at[pl.ds(0, RTAIL)],
                        acc.at[pl.ds(NS * RPT, RTAIL)])
        if with_counts:
            pltpu.sync_copy(z1_v.at[pl.ds(0, RTAIL)],
                            cacc.at[pl.ds(NS * RPT, RTAIL)])

    plsc.subcore_barrier()

    def run(src_hbm, dst_hbm):
        base0 = s * EPT

        def block(i, carry):
            base = base0 + i * (KPB * CHUNK)
            cps = []
            for k in range(KPB):
                cps.append(pltpu.async_copy(
                    src_hbm.at[pl.ds(base + k * CHUNK, CHUNK)], idxs_v[k], isem))
                cps.append(pltpu.async_copy(
                    dst_hbm.at[pl.ds(base + k * CHUNK, CHUNK)], didxs_v[k], isem))
            for cp in cps:
                cp.wait()
            g = [None] * KPB
            scp = [None] * KPB
            for k in range(NB - 1):
                g[k] = pltpu.async_copy(x_hbm.at[idxs_v[k]], rows[k % NB],
                                        gsem[k % NB])
            ccp = []
            for k in range(KPB):
                j = k + NB - 1
                if j < KPB:
                    if j - NB >= 0:
                        scp[j - NB].wait()
                    g[j] = pltpu.async_copy(x_hbm.at[idxs_v[j]], rows[j % NB],
                                            gsem[j % NB])
                g[k].wait()
                scp[k] = pltpu.async_copy(rows[k % NB], acc.at[didxs_v[k]],
                                          ssem[k % NB], add=True)
                if with_counts:
                    ccp.append(pltpu.async_copy(ones_v, cacc.at[didxs_v[k]],
                                                csem, add=True))
            for k in range(KPB - NB, KPB):
                scp[k].wait()
            for cp in ccp:
                cp.wait()
            return carry

        lax.fori_loop(0, NBLK, block, 0)

        base = base0 + NFULL * CHUNK
        pltpu.sync_copy(src_hbm.at[pl.ds(base, TAIL)], idx_t)
        pltpu.sync_copy(dst_hbm.at[pl.ds(base, TAIL)], didx_t)
        pltpu.async_copy(x_hbm.at[idx_t], rows[0].at[pl.ds(0, TAIL)],
                         gsem[0]).wait()
        pltpu.sync_copy(rows[0].at[pl.ds(0, TAIL)], acc.at[didx_t], add=True)
        if with_counts:
            pltpu.sync_copy(ones_v.at[pl.ds(0, TAIL)], cacc.at[didx_t],
                            add=True)

    @pl.when(c == 0)
    def _():
        run(src_c_hbm, dst_c_hbm)

    @pl.when(c == 1)
    def _():
        run(src_r_hbm, dst_r_hbm)

    plsc.subcore_barrier()

    def writeback(sum_hbm, cnt_hbm):
        wcps = []
        o = 0
        for n in _ZSIZES:
            r = row0 + o
            wcps.append(pltpu.async_copy(acc.at[pl.ds(r, n)],
                                         sum_hbm.at[pl.ds(r, n)], isem))
            o += n
        for cp in wcps:
            cp.wait()
        if with_counts:
            o = 0
            for n in _ZSIZES:
                r = row0 + o
                pltpu.sync_copy(cacc.at[pl.ds(r, n)], cbuf_v.at[pl.ds(0, n)])
                pltpu.sync_copy(cbuf_v.at[pl.ds(0, n)], cnt_hbm.at[pl.ds(r, n)])
                o += n

        @pl.when(s == 0)
        def _():
            r = NS * RPT
            pltpu.sync_copy(acc.at[pl.ds(r, RTAIL)], sum_hbm.at[pl.ds(r, RTAIL)])
            if with_counts:
                pltpu.sync_copy(cacc.at[pl.ds(r, RTAIL)],
                                cbuf_v.at[pl.ds(0, RTAIL)])
                pltpu.sync_copy(cbuf_v.at[pl.ds(0, RTAIL)],
                                cnt_hbm.at[pl.ds(r, RTAIL)])

    @pl.when(c == 0)
    def _():
        writeback(sum_c_hbm, cnt_c_hbm if with_counts else None)

    @pl.when(c == 1)
    def _():
        writeback(sum_r_hbm, cnt_r_hbm if with_counts else None)


def _make_agg(with_counts):
    outs = (
        jax.ShapeDtypeStruct((N, D), jnp.float32),
        jax.ShapeDtypeStruct((N, D), jnp.float32),
    )
    if with_counts:
        outs = outs + (
            jax.ShapeDtypeStruct((N,), jnp.float32),
            jax.ShapeDtypeStruct((N,), jnp.float32),
        )
    return pl.kernel(
        functools.partial(_agg_body, with_counts),
        out_type=outs,
        mesh=plsc.VectorSubcoreMesh(core_axis_name="c", subcore_axis_name="s"),
        scratch_types=(
            tuple(pltpu.VMEM((CHUNK,), jnp.int32) for _ in range(KPB))
            + tuple(pltpu.VMEM((CHUNK,), jnp.int32) for _ in range(KPB))
            + tuple(pltpu.VMEM((CHUNK, D), jnp.float32) for _ in range(NB))
            + tuple(pltpu.SemaphoreType.DMA for _ in range(NB))
            + tuple(pltpu.SemaphoreType.DMA for _ in range(NB))
            + (
                pltpu.VMEM_SHARED((N, D), jnp.float32),
                pltpu.VMEM_SHARED((N,), jnp.float32),
                pltpu.VMEM((TAIL,), jnp.int32),
                pltpu.VMEM((TAIL,), jnp.int32),
                pltpu.VMEM((CHUNK,), jnp.float32),
                pltpu.VMEM((CHUNK,), jnp.float32),
                pltpu.VMEM((CHUNK,), jnp.float32),
                pltpu.SemaphoreType.DMA,
                pltpu.SemaphoreType.DMA,
            )
        ),
    )


_agg = _make_agg(True)
_agg_nc = _make_agg(False)


BLK = 2000


def _combine_body(relu, sum_ref, cnt_ref, xd_ref, wl_ref, bl_ref, wr_ref,
                  o_ref):
    cnt = jnp.maximum(cnt_ref[...], 1.0)
    mean = sum_ref[...] / cnt
    acc = lax.dot_general(mean, wl_ref[...], (((1,), (1,)), ((), ())),
                          preferred_element_type=jnp.float32)
    acc = acc + lax.dot_general(xd_ref[...], wr_ref[...],
                                (((1,), (1,)), ((), ())),
                                preferred_element_type=jnp.float32)
    acc = acc + bl_ref[...]
    if relu:
        acc = jnp.maximum(acc, 0.0)
    o_ref[...] = acc


def _make_combine(relu):
    return pl.pallas_call(
        functools.partial(_combine_body, relu),
        out_shape=jax.ShapeDtypeStruct((N, D), jnp.float32),
        grid=(N // BLK,),
        in_specs=[
            pl.BlockSpec((BLK, D), lambda i: (i, 0)),
            pl.BlockSpec((BLK, 1), lambda i: (i, 0)),
            pl.BlockSpec((BLK, D), lambda i: (i, 0)),
            pl.BlockSpec((D, D), lambda i: (0, 0)),
            pl.BlockSpec((1, D), lambda i: (0, 0)),
            pl.BlockSpec((D, D), lambda i: (0, 0)),
        ],
        out_specs=pl.BlockSpec((BLK, D), lambda i: (i, 0)),
    )


_combine_relu = _make_combine(True)
_combine_lin = _make_combine(False)


def kernel(x_paper, x_author, edge_index_cites, edge_index_rev_writes,
           Wl1c, bl1c, Wr1c, Wl1r, bl1r, Wr1r,
           Wl2c, bl2c, Wr2c, Wl2r, bl2r, Wr2r):
    src_c = edge_index_cites[0].astype(jnp.int32)
    dst_c = edge_index_cites[1].astype(jnp.int32)
    src_r = edge_index_rev_writes[0].astype(jnp.int32)
    dst_r = edge_index_rev_writes[1].astype(jnp.int32)

    sum_c1, sum_r1, cnt_c, cnt_r = _agg(x_paper, src_c, dst_c, src_r, dst_r)
    cnt_c = cnt_c[:, None]
    cnt_r = cnt_r[:, None]
    p1 = _combine_relu(sum_c1, cnt_c, x_paper,
                       Wl1c, bl1c.reshape(1, D), Wr1c)
    a1 = _combine_relu(sum_r1, cnt_r, x_author,
                       Wl1r, bl1r.reshape(1, D), Wr1r)

    sum_c2, sum_r2 = _agg_nc(p1, src_c, dst_c, src_r, dst_r)
    p2 = _combine_lin(sum_c2, cnt_c, p1, Wl2c, bl2c.reshape(1, D), Wr2c)
    a2 = _combine_lin(sum_r2, cnt_r, a1, Wl2r, bl2r.reshape(1, D), Wr2r)
    return (p2, a2)
